# Initial kernel scaffold; baseline (speedup 1.0000x reference)
import jax
import jax.numpy as jnp
from jax.experimental import pallas as pl


def kernel(feat, segment_ids, num_segments):
    raise NotImplementedError("write your pallas kernel here")


# speedup vs baseline: 3.4671x; 3.4671x over previous
import functools

import jax
import jax.numpy as jnp
from jax import lax
from jax.experimental import pallas as pl
from jax.experimental.pallas import tpu as pltpu
from jax.experimental.pallas import tpu_sc as plsc

_N_ROWS = 100000
_D = 128
_NSEG = 512
_NC = 2
_NS = 16
_NW = _NC * _NS
_ROWS_PER_W = _N_ROWS // _NW
_CHUNK = 125
_NCHUNK = _ROWS_PER_W // _CHUNK
_ZROWS = _NSEG // _NS


def _sc_partial_segment_sum(feat, ids2d):
    mesh = plsc.VectorSubcoreMesh(core_axis_name="c", subcore_axis_name="s")

    @functools.partial(
        pl.kernel,
        mesh=mesh,
        out_type=jax.ShapeDtypeStruct((_NC * _NSEG, _D), jnp.float32),
        scratch_types=[
            pltpu.VMEM_SHARED((_NSEG, _D), jnp.float32),
            pltpu.VMEM((_NCHUNK, _CHUNK), jnp.int32),
            pltpu.VMEM((_CHUNK, _D), jnp.float32),
            pltpu.VMEM((_CHUNK, _D), jnp.float32),
            pltpu.SemaphoreType.DMA,
            pltpu.SemaphoreType.DMA,
        ],
    )
    def k(feat_hbm, ids_hbm, out_hbm, acc, idx_v, buf0, buf1, sem0, sem1):
        cid = lax.axis_index("c")
        sid = lax.axis_index("s")
        wid = cid * _NS + sid

        pltpu.sync_copy(ids_hbm.at[wid], idx_v)

        zero = jnp.zeros((16,), jnp.float32)
        for r in range(_ZROWS):
            for l in range(_D // 16):
                buf0[r, pl.ds(l * 16, 16)] = zero
        pltpu.sync_copy(buf0.at[pl.ds(0, _ZROWS)],
                        acc.at[pl.ds(sid * _ZROWS, _ZROWS)])
        plsc.subcore_barrier()

        bufs = (buf0, buf1)
        sems = (sem0, sem1)
        pending = pltpu.async_copy(feat_hbm.at[wid * _NCHUNK], buf0, sem0)
        for j in range(_NCHUNK):
            pending.wait()
            if j + 1 < _NCHUNK:
                pending = pltpu.async_copy(
                    feat_hbm.at[wid * _NCHUNK + j + 1],
                    bufs[(j + 1) % 2], sems[(j + 1) % 2])
            pltpu.sync_copy(bufs[j % 2], acc.at[idx_v.at[j]], add=True)

        plsc.subcore_barrier()
        pltpu.sync_copy(acc.at[pl.ds(sid * _ZROWS, _ZROWS)],
                        out_hbm.at[pl.ds(cid * _NSEG + sid * _ZROWS, _ZROWS)])

    return k(feat, ids2d)


def _tc_merge(x_ref, o_ref):
    o_ref[...] = x_ref[0] + x_ref[1]


def kernel(feat, segment_ids, num_segments):
    ids = segment_ids.astype(jnp.int32).reshape(_NW, _NCHUNK, _CHUNK)
    feat3 = feat.reshape(_NW * _NCHUNK, _CHUNK, _D)
    part = _sc_partial_segment_sum(feat3, ids).reshape(_NC, _NSEG, _D)
    return pl.pallas_call(
        _tc_merge,
        out_shape=jax.ShapeDtypeStruct((_NSEG, _D), jnp.float32),
    )(part)


# speedup vs baseline: 6.1667x; 1.7786x over previous
import functools

import jax
import jax.numpy as jnp
from jax import lax
from jax.experimental import pallas as pl
from jax.experimental.pallas import tpu as pltpu
from jax.experimental.pallas import tpu_sc as plsc

_N_ROWS = 100000
_D = 128
_NSEG = 512
_NC = 2
_NS = 16
_NW = _NC * _NS
_CHUNK = 128
_NFULL = _N_ROWS // _CHUNK
_TAIL = _N_ROWS - _NFULL * _CHUNK
_NJ = _NFULL // _NW
_NEXTRA = _NFULL - _NJ * _NW
_ZROWS = _NSEG // _NS


def _sc_partial_segment_sum(feat, ids):
    mesh = plsc.VectorSubcoreMesh(core_axis_name="c", subcore_axis_name="s")

    @functools.partial(
        pl.kernel,
        mesh=mesh,
        out_type=jax.ShapeDtypeStruct((_NC * _NSEG, _D), jnp.float32),
        scratch_types=[
            pltpu.VMEM_SHARED((_NSEG, _D), jnp.float32),
            pltpu.VMEM((_CHUNK, _D), jnp.float32),
            pltpu.VMEM((_CHUNK, _D), jnp.float32),
            pltpu.VMEM((_CHUNK,), jnp.int32),
            pltpu.VMEM((_CHUNK,), jnp.int32),
            pltpu.VMEM((_TAIL, _D), jnp.float32),
            pltpu.VMEM((_TAIL,), jnp.int32),
            pltpu.SemaphoreType.DMA,
            pltpu.SemaphoreType.DMA,
        ],
    )
    def k(feat_hbm, ids_hbm, out_hbm, acc,
          buf0, buf1, idx0, idx1, tbuf, tidx, sem0, sem1):
        cid = lax.axis_index("c")
        sid = lax.axis_index("s")
        wid = cid * _NS + sid

        zero = jnp.zeros((16,), jnp.float32)
        for r in range(_ZROWS):
            for l in range(_D // 16):
                buf0[r, pl.ds(l * 16, 16)] = zero
        pltpu.sync_copy(buf0.at[pl.ds(0, _ZROWS)],
                        acc.at[pl.ds(sid * _ZROWS, _ZROWS)])
        plsc.subcore_barrier()

        bufs = (buf0, buf1)
        idxs = (idx0, idx1)
        sems = (sem0, sem1)

        def start_load(j, slot):
            row = pl.multiple_of((j * _NW + wid) * _CHUNK, _CHUNK)
            f = pltpu.async_copy(feat_hbm.at[pl.ds(row, _CHUNK)],
                                 bufs[slot], sems[slot])
            i = pltpu.async_copy(ids_hbm.at[pl.ds(row, _CHUNK)],
                                 idxs[slot], sems[slot])
            return f, i

        pending = start_load(0, 0)
        for j in range(_NJ):
            for h in pending:
                h.wait()
            if j + 1 < _NJ:
                pending = start_load(j + 1, (j + 1) % 2)
            pltpu.sync_copy(bufs[j % 2], acc.at[idxs[j % 2]], add=True)

        @pl.when(wid < _NEXTRA)
        def _():
            for h in start_load(_NJ, _NJ % 2):
                h.wait()
            pltpu.sync_copy(bufs[_NJ % 2], acc.at[idxs[_NJ % 2]], add=True)

        @pl.when(wid == _NW - 1)
        def _():
            base = _NFULL * _CHUNK
            f = pltpu.async_copy(feat_hbm.at[pl.ds(base, _TAIL)], tbuf, sem0)
            i = pltpu.async_copy(ids_hbm.at[pl.ds(base, _TAIL)], tidx, sem1)
            f.wait()
            i.wait()
            pltpu.sync_copy(tbuf, acc.at[tidx], add=True)

        plsc.subcore_barrier()
        pltpu.sync_copy(acc.at[pl.ds(sid * _ZROWS, _ZROWS)],
                        out_hbm.at[pl.ds(cid * _NSEG + sid * _ZROWS, _ZROWS)])

    return k(feat, ids)


def _tc_merge(x_ref, o_ref):
    o_ref[...] = x_ref[0] + x_ref[1]


def kernel(feat, segment_ids, num_segments):
    ids = segment_ids.astype(jnp.int32)
    part = _sc_partial_segment_sum(feat, ids).reshape(_NC, _NSEG, _D)
    return pl.pallas_call(
        _tc_merge,
        out_shape=jax.ShapeDtypeStruct((_NSEG, _D), jnp.float32),
    )(part)


# speedup vs baseline: 6.7781x; 1.0991x over previous
import functools

import jax
import jax.numpy as jnp
from jax import lax
from jax.experimental import pallas as pl
from jax.experimental.pallas import tpu as pltpu
from jax.experimental.pallas import tpu_sc as plsc

_N_ROWS = 100000
_D = 128
_NSEG = 512
_NC = 2
_NS = 16
_NW = _NC * _NS
_CHUNK = 128
_NFULL = _N_ROWS // _CHUNK
_TAIL = _N_ROWS - _NFULL * _CHUNK
_NJ = _NFULL // _NW
_NEXTRA = _NFULL - _NJ * _NW
_ZROWS = _NSEG // _NS


def _sc_partial_segment_sum(feat, ids):
    mesh = plsc.VectorSubcoreMesh(core_axis_name="c", subcore_axis_name="s")

    @functools.partial(
        pl.kernel,
        mesh=mesh,
        out_type=jax.ShapeDtypeStruct((_NC * _NSEG, _D), jnp.float32),
        scratch_types=(
            [pltpu.VMEM_SHARED((_NSEG, _D), jnp.float32)]
            + [pltpu.VMEM((_CHUNK, _D), jnp.float32)] * 4
            + [pltpu.VMEM((_CHUNK,), jnp.int32)] * 4
            + [pltpu.VMEM((_TAIL, _D), jnp.float32),
               pltpu.VMEM((_TAIL,), jnp.int32)]
            + [pltpu.SemaphoreType.DMA] * 8
        ),
    )
    def k(feat_hbm, ids_hbm, out_hbm, acc,
          buf0, buf1, buf2, buf3, idx0, idx1, idx2, idx3, tbuf, tidx,
          ls0, ls1, ls2, ls3, ss0, ss1, ss2, ss3):
        cid = lax.axis_index("c")
        sid = lax.axis_index("s")
        wid = cid * _NS + sid
        bufs = (buf0, buf1, buf2, buf3)
        idxs = (idx0, idx1, idx2, idx3)
        lsems = (ls0, ls1, ls2, ls3)
        ssems = (ss0, ss1, ss2, ss3)

        zero = jnp.zeros((16,), jnp.float32)
        for r in range(_ZROWS):
            for l in range(_D // 16):
                buf0[r, pl.ds(l * 16, 16)] = zero
        pltpu.sync_copy(buf0.at[pl.ds(0, _ZROWS)],
                        acc.at[pl.ds(sid * _ZROWS, _ZROWS)])

        def start_load(j):
            s = j % 4
            row = pl.multiple_of((j * _NW + wid) * _CHUNK, _CHUNK)
            f = pltpu.async_copy(feat_hbm.at[pl.ds(row, _CHUNK)],
                                 bufs[s], lsems[s])
            i = pltpu.async_copy(ids_hbm.at[pl.ds(row, _CHUNK)],
                                 idxs[s], lsems[s])
            return f, i

        loads = {0: start_load(0), 1: start_load(1)}
        plsc.subcore_barrier()
        scats = {}
        for j in range(_NJ):
            if j >= 2:
                scats.pop(j - 2).wait()
            if j + 2 < _NJ:
                loads[j + 2] = start_load(j + 2)
            for h in loads.pop(j):
                h.wait()
            scats[j] = pltpu.async_copy(bufs[j % 4], acc.at[idxs[j % 4]],
                                        ssems[j % 4], add=True)
        for j in sorted(scats):
            scats.pop(j).wait()

        @pl.when(wid < _NEXTRA)
        def _():
            for h in start_load(_NJ):
                h.wait()
            pltpu.sync_copy(bufs[_NJ % 4], acc.at[idxs[_NJ % 4]], add=True)

        @pl.when(wid == _NW - 1)
        def _():
            base = _NFULL * _CHUNK
            f = pltpu.async_copy(feat_hbm.at[pl.ds(base, _TAIL)], tbuf, ls0)
            i = pltpu.async_copy(ids_hbm.at[pl.ds(base, _TAIL)], tidx, ls1)
            f.wait()
            i.wait()
            pltpu.sync_copy(tbuf, acc.at[tidx], add=True)

        plsc.subcore_barrier()
        pltpu.sync_copy(acc.at[pl.ds(sid * _ZROWS, _ZROWS)],
                        out_hbm.at[pl.ds(cid * _NSEG + sid * _ZROWS, _ZROWS)])

    return k(feat, ids)


def _tc_merge(x_ref, o_ref):
    o_ref[...] = x_ref[0] + x_ref[1]


def kernel(feat, segment_ids, num_segments):
    ids = segment_ids.astype(jnp.int32)
    part = _sc_partial_segment_sum(feat, ids).reshape(_NC, _NSEG, _D)
    return pl.pallas_call(
        _tc_merge,
        out_shape=jax.ShapeDtypeStruct((_NSEG, _D), jnp.float32),
    )(part)
